# baseline (device time: 26762 ns/iter reference)
import jax
import jax.numpy as jnp
from jax import lax
from jax.experimental import pallas as pl
from jax.experimental.pallas import tpu as pltpu

N_DEV = 8
M = 1024
N = 1024
K = 4
H = M // 2 // K
N_WINDOWS = 4
N_STEPS = K * N_WINDOWS

PARTS = [
    (0, 384, (0, 1, 2)),
    (384, 384, (1, 2, 0)),
    (768, 256, (2, 0, 1)),
]


def _coords(my):
    q = my % 4
    cx = jnp.where((q == 1) | (q == 2), 1, 0).astype(jnp.int32)
    cy = q // 2
    cz = my // 4
    return (cx, cy, cz)


def _partner(my, axis):
    q = my % 4
    if axis == 0:
        return my + 1 - 2 * (q % 2)
    if axis == 1:
        return my - 2 * q + 3
    return (my + 4) % N_DEV


def kernel(x, w_mat):
    n_parts = len(PARTS)

    def body(x_ref, w_ref, out_ref, *scratch):
        bufs = scratch[: 2 * N_WINDOWS * n_parts]
        send_sems, recv_sems = scratch[2 * N_WINDOWS * n_parts:]

        def sb(p, w):
            return bufs[2 * N_WINDOWS * p + 2 * w]

        def rb(p, w):
            return bufs[2 * N_WINDOWS * p + 2 * w + 1]

        my = lax.axis_index("i")
        coords = _coords(my)
        all_rdmas = []

        def exchange(p, w, h, src, dst, axis):
            rdma = pltpu.make_async_remote_copy(
                src_ref=src,
                dst_ref=dst,
                send_sem=send_sems.at[N_STEPS * p + K * w + h],
                recv_sem=recv_sems.at[N_STEPS * p + K * w + h],
                device_id=_partner(my, axis),
                device_id_type=pl.DeviceIdType.LOGICAL,
            )
            all_rdmas.append(rdma)
            rdma.start()
            return rdma

        barrier = pltpu.get_barrier_semaphore()
        for axis in range(3):
            pl.semaphore_signal(
                barrier,
                inc=1,
                device_id=_partner(my, axis),
                device_id_type=pl.DeviceIdType.LOGICAL,
            )
        pl.semaphore_wait(barrier, 3)

        keep0 = []
        send0 = []
        for p, (c0, nc, order) in enumerate(PARTS):
            c = coords[order[0]]
            keep0.append(c * (M // 2))
            send0.append((1 - c) * (M // 2))

        rdmas = {}

        for p, (c0, nc, order) in enumerate(PARTS):
            part_send = jnp.dot(
                x_ref[pl.ds(send0[p], M // 2), :].astype(jnp.bfloat16),
                w_ref[:, pl.ds(c0, nc)].astype(jnp.bfloat16),
                preferred_element_type=jnp.float32,
            )
            sb(p, 0)[:, :] = part_send.astype(jnp.bfloat16)
            for h in range(K):
                rdmas[(p, 0, h)] = exchange(
                    p,
                    0,
                    h,
                    sb(p, 0).at[pl.ds(h * H, H), :],
                    rb(p, 0).at[pl.ds(h * H, H), :],
                    order[0],
                )

        for p, (c0, nc, order) in enumerate(PARTS):
            out_ref[pl.ds(keep0[p], M // 2), pl.ds(c0, nc)] = jnp.dot(
                x_ref[pl.ds(keep0[p], M // 2), :].astype(jnp.bfloat16),
                w_ref[:, pl.ds(c0, nc)].astype(jnp.bfloat16),
                preferred_element_type=jnp.float32,
            )

        for w in range(3):
            for h in range(K):
                for p, (c0, nc, order) in enumerate(PARTS):
                    rows = pl.ds(keep0[p] + h * H, H)
                    rel = pl.ds(h * H, H)
                    rdmas[(p, w, h)].wait_recv()
                    val = out_ref[rows, pl.ds(c0, nc)] + rb(p, w)[
                        rel, :
                    ].astype(jnp.float32)
                    sb(p, w + 1)[rel, :] = val.astype(jnp.bfloat16)
                    rdmas[(p, w + 1, h)] = exchange(
                        p,
                        w + 1,
                        h,
                        sb(p, w + 1).at[rel, :],
                        rb(p, w + 1).at[rel, :],
                        order[(w + 1) % 3],
                    )
                    out_ref[rows, pl.ds(c0, nc)] = val

        for h in range(K):
            for p, (c0, nc, order) in enumerate(PARTS):
                rdmas[(p, 3, h)].wait_recv()
                out_ref[pl.ds(send0[p] + h * H, H), pl.ds(c0, nc)] = rb(p, 3)[
                    pl.ds(h * H, H), :
                ].astype(jnp.float32)

        for r in all_rdmas:
            r.wait_send()

    scratch_shapes = []
    for (c0, nc, order) in PARTS:
        scratch_shapes += [pltpu.VMEM((M // 2, nc), jnp.bfloat16)] * (2 * N_WINDOWS)
    n_sems = n_parts * N_STEPS
    scratch_shapes += [
        pltpu.SemaphoreType.DMA((n_sems,)),
        pltpu.SemaphoreType.DMA((n_sems,)),
    ]

    return pl.pallas_call(
        body,
        out_shape=jax.ShapeDtypeStruct((M, N), jnp.float32),
        in_specs=[
            pl.BlockSpec(memory_space=pltpu.VMEM),
            pl.BlockSpec(memory_space=pltpu.VMEM),
        ],
        out_specs=pl.BlockSpec(memory_space=pltpu.VMEM),
        scratch_shapes=scratch_shapes,
        compiler_params=pltpu.CompilerParams(collective_id=0),
    )(x, w_mat)
